# baseline (speedup 1.0000x reference)
import functools

import jax
import jax.numpy as jnp
from jax import lax
from jax.experimental import pallas as pl
from jax.experimental.pallas import tpu as pltpu
from jax.experimental.pallas import tpu_sc as plsc

B = 16384
D = 64
D_TYPE = 32
D_VAR = 16
H = 128
V_TYPE = 100
V_VAR = 3

NC = 2
NS = 16
NW = NC * NS
B_PER_W = B // NW
GATHER_CHUNK = 128

BLK = 1024
GRID = B // BLK


def _sc_gather(table, idx):
    mesh = plsc.VectorSubcoreMesh(core_axis_name="c", subcore_axis_name="s")

    @functools.partial(
        pl.kernel,
        mesh=mesh,
        out_type=jax.ShapeDtypeStruct((B, D), jnp.float32),
        scratch_types=[
            pltpu.VMEM((B_PER_W,), jnp.int32),
            pltpu.VMEM((B_PER_W, D), jnp.float32),
            pltpu.SemaphoreType.DMA,
        ],
        compiler_params=pltpu.CompilerParams(use_tc_tiling_on_sc=False),
    )
    def k(table_hbm, idx_hbm, out_hbm, idx_v, rows_v, sem):
        wid = lax.axis_index("s") * NC + lax.axis_index("c")
        base = wid * B_PER_W
        pltpu.sync_copy(idx_hbm.at[pl.ds(base, B_PER_W)], idx_v)
        copies = []
        for j in range(B_PER_W // GATHER_CHUNK):
            copies.append(pltpu.async_copy(
                table_hbm.at[idx_v.at[pl.ds(j * GATHER_CHUNK, GATHER_CHUNK)]],
                rows_v.at[pl.ds(j * GATHER_CHUNK, GATHER_CHUNK), :],
                sem,
            ))
        for c in copies:
            c.wait()
        pltpu.sync_copy(rows_v, out_hbm.at[pl.ds(base, B_PER_W)])

    return k(table, idx)


def _mlp_body(t_ref, v_ref, pe_ref, tt_ref, tv_ref, w1_ref, b1_ref, w2_ref,
              b2_ref, out_ref):
    tids = t_ref[0, 0, :]
    vids = v_ref[0, 0, :]
    pe = pe_ref[...]

    iota_t = lax.broadcasted_iota(jnp.int32, (BLK, V_TYPE), 1)
    onehot_t = (tids[:, None] == iota_t).astype(jnp.float32)
    te = jnp.dot(onehot_t, tt_ref[...], preferred_element_type=jnp.float32)

    ve = ((vids[:, None] == 0).astype(jnp.float32) * tv_ref[0:1, :]
          + (vids[:, None] == 1).astype(jnp.float32) * tv_ref[1:2, :]
          + (vids[:, None] == 2).astype(jnp.float32) * tv_ref[2:3, :])

    h = (jnp.dot(pe, w1_ref[0:D, :], preferred_element_type=jnp.float32)
         + jnp.dot(te, w1_ref[D:D + D_TYPE, :],
                   preferred_element_type=jnp.float32)
         + jnp.dot(ve, w1_ref[D + D_TYPE:D + D_TYPE + D_VAR, :],
                   preferred_element_type=jnp.float32)
         + b1_ref[:][None, :])
    h = jnp.maximum(h, 0.0)
    out_ref[...] = (jnp.dot(h, w2_ref[...], preferred_element_type=jnp.float32)
                    + b2_ref[:][None, :])


def _mlp(pe, t3, v3, table_type, table_var, W1, b1, W2, b2):
    return pl.pallas_call(
        _mlp_body,
        grid=(GRID,),
        in_specs=[
            pl.BlockSpec((1, 1, BLK), lambda i: (i, 0, 0)),
            pl.BlockSpec((1, 1, BLK), lambda i: (i, 0, 0)),
            pl.BlockSpec((BLK, D), lambda i: (i, 0)),
            pl.BlockSpec((V_TYPE, D_TYPE), lambda i: (0, 0)),
            pl.BlockSpec((V_VAR, D_VAR), lambda i: (0, 0)),
            pl.BlockSpec((D + D_TYPE + D_VAR, H), lambda i: (0, 0)),
            pl.BlockSpec((H,), lambda i: (0,)),
            pl.BlockSpec((H, D), lambda i: (0, 0)),
            pl.BlockSpec((D,), lambda i: (0,)),
        ],
        out_specs=pl.BlockSpec((BLK, D), lambda i: (i, 0)),
        out_shape=jax.ShapeDtypeStruct((B, D), jnp.float32),
    )(t3, v3, pe, table_type, table_var, W1, b1, W2, b2)


def kernel(product_id, product_type, variant_sellable, table_prod, table_type,
           table_var, W1, b1, W2, b2):
    pe = _sc_gather(table_prod, product_id)
    t3 = product_type.astype(jnp.int32).reshape(GRID, 1, BLK)
    v3 = variant_sellable.astype(jnp.int32).reshape(GRID, 1, BLK)
    return _mlp(pe, t3, v3, table_type, table_var, W1, b1, W2, b2)


# speedup vs baseline: 2.1225x; 2.1225x over previous
import functools

import jax
import jax.numpy as jnp
from jax import lax
from jax.experimental import pallas as pl
from jax.experimental.pallas import tpu as pltpu
from jax.experimental.pallas import tpu_sc as plsc

B = 16384
D = 64
D_TYPE = 32
D_VAR = 16
H = 128
V_TYPE = 100
V_VAR = 3

NC = 2
NS = 16
NW = NC * NS
B_PER_W = B // NW
GATHER_CHUNK = 128
N_CHUNK = B_PER_W // GATHER_CHUNK
LANES = 16

BLK = 1024
GRID = B // BLK


TBLK = 8192
HBLK = TBLK // 2
TGRID = (1000000 + TBLK - 1) // TBLK
NPAIR = TGRID * HBLK


def _transpose_body(in_ref, out_ref):
    out_ref[:, 0:D] = in_ref[:, 0:HBLK].T
    out_ref[:, D:2 * D] = in_ref[:, HBLK:TBLK].T


def _transpose_pack(table_t):
    return pl.pallas_call(
        _transpose_body,
        grid=(TGRID,),
        in_specs=[pl.BlockSpec((D, TBLK), lambda i: (0, i))],
        out_specs=pl.BlockSpec((HBLK, 2 * D), lambda i: (i, 0)),
        out_shape=jax.ShapeDtypeStruct((NPAIR, 2 * D), jnp.float32),
    )(table_t)


def _sc_gather(table, idx):
    mesh = plsc.VectorSubcoreMesh(core_axis_name="c", subcore_axis_name="s")

    @functools.partial(
        pl.kernel,
        mesh=mesh,
        out_type=jax.ShapeDtypeStruct((B, D), jnp.float32),
        scratch_types=[
            pltpu.VMEM((B_PER_W,), jnp.int32),
            pltpu.VMEM((B_PER_W,), jnp.int32),
            pltpu.VMEM((B_PER_W, D), jnp.float32),
            pltpu.SemaphoreType.DMA,
        ],
        compiler_params=pltpu.CompilerParams(use_tc_tiling_on_sc=False),
    )
    def k(table_hbm, idx_hbm, out_hbm, idx_v, j_v, rows_v, sem):
        wid = lax.axis_index("s") * NC + lax.axis_index("c")
        base = wid * B_PER_W
        pltpu.sync_copy(idx_hbm.at[pl.ds(base, B_PER_W)], idx_v)

        def transform(kk, _):
            v = idx_v[pl.ds(kk * LANES, LANES)]
            g = ((v >> 13) << 12) + (v & (HBLK - 1))
            j_v[pl.ds(kk * LANES, LANES)] = (g << 1) | ((v >> 12) & 1)
            return 0

        lax.fori_loop(0, B_PER_W // LANES, transform, 0)
        copies = []
        for c in range(N_CHUNK):
            copies.append(pltpu.async_copy(
                table_hbm.at[j_v.at[pl.ds(c * GATHER_CHUNK, GATHER_CHUNK)]],
                rows_v.at[pl.ds(c * GATHER_CHUNK, GATHER_CHUNK), :],
                sem,
            ))
        for cp in copies:
            cp.wait()
        pltpu.sync_copy(rows_v, out_hbm.at[pl.ds(base, B_PER_W)])

    return k(table, idx)


def _mlp_body(t_ref, v_ref, pe_ref, tt_ref, tv_ref, w1_ref, b1_ref, w2_ref,
              b2_ref, out_ref):
    tids = t_ref[0, 0, :]
    vids = v_ref[0, 0, :]
    pe = pe_ref[...]

    iota_t = lax.broadcasted_iota(jnp.int32, (BLK, V_TYPE), 1)
    onehot_t = (tids[:, None] == iota_t).astype(jnp.float32)
    te = jnp.dot(onehot_t, tt_ref[...], preferred_element_type=jnp.float32)

    ve = ((vids[:, None] == 0).astype(jnp.float32) * tv_ref[0:1, :]
          + (vids[:, None] == 1).astype(jnp.float32) * tv_ref[1:2, :]
          + (vids[:, None] == 2).astype(jnp.float32) * tv_ref[2:3, :])

    h = (jnp.dot(pe, w1_ref[0:D, :], preferred_element_type=jnp.float32)
         + jnp.dot(te, w1_ref[D:D + D_TYPE, :],
                   preferred_element_type=jnp.float32)
         + jnp.dot(ve, w1_ref[D + D_TYPE:D + D_TYPE + D_VAR, :],
                   preferred_element_type=jnp.float32)
         + b1_ref[:][None, :])
    h = jnp.maximum(h, 0.0)
    out_ref[...] = (jnp.dot(h, w2_ref[...], preferred_element_type=jnp.float32)
                    + b2_ref[:][None, :])


def _mlp(pe, t3, v3, table_type, table_var, W1, b1, W2, b2):
    return pl.pallas_call(
        _mlp_body,
        grid=(GRID,),
        in_specs=[
            pl.BlockSpec((1, 1, BLK), lambda i: (i, 0, 0)),
            pl.BlockSpec((1, 1, BLK), lambda i: (i, 0, 0)),
            pl.BlockSpec((BLK, D), lambda i: (i, 0)),
            pl.BlockSpec((V_TYPE, D_TYPE), lambda i: (0, 0)),
            pl.BlockSpec((V_VAR, D_VAR), lambda i: (0, 0)),
            pl.BlockSpec((D + D_TYPE + D_VAR, H), lambda i: (0, 0)),
            pl.BlockSpec((H,), lambda i: (0,)),
            pl.BlockSpec((H, D), lambda i: (0, 0)),
            pl.BlockSpec((D,), lambda i: (0,)),
        ],
        out_specs=pl.BlockSpec((BLK, D), lambda i: (i, 0)),
        out_shape=jax.ShapeDtypeStruct((B, D), jnp.float32),
    )(t3, v3, pe, table_type, table_var, W1, b1, W2, b2)


def kernel(product_id, product_type, variant_sellable, table_prod, table_type,
           table_var, W1, b1, W2, b2):
    pairs = _transpose_pack(table_prod.T)
    table_rm = jnp.reshape(pairs, (2 * NPAIR, D))
    pe = _sc_gather(table_rm, product_id)
    t3 = product_type.astype(jnp.int32).reshape(GRID, 1, BLK)
    v3 = variant_sellable.astype(jnp.int32).reshape(GRID, 1, BLK)
    return _mlp(pe, t3, v3, table_type, table_var, W1, b1, W2, b2)


# speedup vs baseline: 2.4046x; 1.1329x over previous
import functools

import jax
import jax.numpy as jnp
from jax import lax
from jax.experimental import pallas as pl
from jax.experimental.pallas import tpu as pltpu
from jax.experimental.pallas import tpu_sc as plsc

B = 16384
D = 64
D_TYPE = 32
D_VAR = 16
H = 128
V_TYPE = 100
V_VAR = 3

NC = 2
NS = 16
NW = NC * NS
B_PER_W = B // NW
GATHER_CHUNK = 128
N_CHUNK = B_PER_W // GATHER_CHUNK
LANES = 16

BLK = 2048
GRID = B // BLK


TBLK = 16384
HBLK = TBLK // 2
TGRID = (1000000 + TBLK - 1) // TBLK
NPAIR = TGRID * HBLK
TSHIFT = TBLK.bit_length() - 1


def _transpose_body(in_ref, out_ref):
    out_ref[:, 0:D] = in_ref[:, 0:HBLK].T
    out_ref[:, D:2 * D] = in_ref[:, HBLK:TBLK].T


def _transpose_pack(table_t):
    return pl.pallas_call(
        _transpose_body,
        grid=(TGRID,),
        in_specs=[pl.BlockSpec((D, TBLK), lambda i: (0, i))],
        out_specs=pl.BlockSpec((HBLK, 2 * D), lambda i: (i, 0)),
        out_shape=jax.ShapeDtypeStruct((NPAIR, 2 * D), jnp.float32),
    )(table_t)


def _sc_gather(table, idx):
    mesh = plsc.VectorSubcoreMesh(core_axis_name="c", subcore_axis_name="s")

    @functools.partial(
        pl.kernel,
        mesh=mesh,
        out_type=jax.ShapeDtypeStruct((B, D), jnp.float32),
        scratch_types=[
            pltpu.VMEM((B_PER_W,), jnp.int32),
            pltpu.VMEM((B_PER_W,), jnp.int32),
            pltpu.VMEM((B_PER_W, D), jnp.float32),
            pltpu.SemaphoreType.DMA,
        ],
        compiler_params=pltpu.CompilerParams(use_tc_tiling_on_sc=False),
    )
    def k(table_hbm, idx_hbm, out_hbm, idx_v, j_v, rows_v, sem):
        wid = lax.axis_index("s") * NC + lax.axis_index("c")
        base = wid * B_PER_W
        pltpu.sync_copy(idx_hbm.at[pl.ds(base, B_PER_W)], idx_v)

        def transform(kk, _):
            v = idx_v[pl.ds(kk * LANES, LANES)]
            g = ((v >> TSHIFT) << (TSHIFT - 1)) + (v & (HBLK - 1))
            j_v[pl.ds(kk * LANES, LANES)] = (g << 1) | ((v >> (TSHIFT - 1)) & 1)
            return 0

        lax.fori_loop(0, B_PER_W // LANES, transform, 0)
        copies = []
        for c in range(N_CHUNK):
            copies.append(pltpu.async_copy(
                table_hbm.at[j_v.at[pl.ds(c * GATHER_CHUNK, GATHER_CHUNK)]],
                rows_v.at[pl.ds(c * GATHER_CHUNK, GATHER_CHUNK), :],
                sem,
            ))
        for cp in copies:
            cp.wait()
        pltpu.sync_copy(rows_v, out_hbm.at[pl.ds(base, B_PER_W)])

    return k(table, idx)


def _mlp_body(t_ref, v_ref, pe_ref, tt_ref, tv_ref, w1_ref, b1_ref, w2_ref,
              b2_ref, out_ref):
    tids = t_ref[0, 0, :]
    vids = v_ref[0, 0, :]
    pe = pe_ref[...]

    iota_t = lax.broadcasted_iota(jnp.int32, (BLK, V_TYPE), 1)
    onehot_t = (tids[:, None] == iota_t).astype(jnp.float32)
    te = jnp.dot(onehot_t, tt_ref[...], preferred_element_type=jnp.float32)

    ve = ((vids[:, None] == 0).astype(jnp.float32) * tv_ref[0:1, :]
          + (vids[:, None] == 1).astype(jnp.float32) * tv_ref[1:2, :]
          + (vids[:, None] == 2).astype(jnp.float32) * tv_ref[2:3, :])

    h = (jnp.dot(pe, w1_ref[0:D, :], preferred_element_type=jnp.float32)
         + jnp.dot(te, w1_ref[D:D + D_TYPE, :],
                   preferred_element_type=jnp.float32)
         + jnp.dot(ve, w1_ref[D + D_TYPE:D + D_TYPE + D_VAR, :],
                   preferred_element_type=jnp.float32)
         + b1_ref[:][None, :])
    h = jnp.maximum(h, 0.0)
    out_ref[...] = (jnp.dot(h, w2_ref[...], preferred_element_type=jnp.float32)
                    + b2_ref[:][None, :])


def _mlp(pe, t3, v3, table_type, table_var, W1, b1, W2, b2):
    return pl.pallas_call(
        _mlp_body,
        grid=(GRID,),
        in_specs=[
            pl.BlockSpec((1, 1, BLK), lambda i: (i, 0, 0)),
            pl.BlockSpec((1, 1, BLK), lambda i: (i, 0, 0)),
            pl.BlockSpec((BLK, D), lambda i: (i, 0)),
            pl.BlockSpec((V_TYPE, D_TYPE), lambda i: (0, 0)),
            pl.BlockSpec((V_VAR, D_VAR), lambda i: (0, 0)),
            pl.BlockSpec((D + D_TYPE + D_VAR, H), lambda i: (0, 0)),
            pl.BlockSpec((H,), lambda i: (0,)),
            pl.BlockSpec((H, D), lambda i: (0, 0)),
            pl.BlockSpec((D,), lambda i: (0,)),
        ],
        out_specs=pl.BlockSpec((BLK, D), lambda i: (i, 0)),
        out_shape=jax.ShapeDtypeStruct((B, D), jnp.float32),
    )(t3, v3, pe, table_type, table_var, W1, b1, W2, b2)


def kernel(product_id, product_type, variant_sellable, table_prod, table_type,
           table_var, W1, b1, W2, b2):
    pairs = _transpose_pack(table_prod.T)
    table_rm = jnp.reshape(pairs, (2 * NPAIR, D))
    pe = _sc_gather(table_rm, product_id)
    t3 = product_type.astype(jnp.int32).reshape(GRID, 1, BLK)
    v3 = variant_sellable.astype(jnp.int32).reshape(GRID, 1, BLK)
    return _mlp(pe, t3, v3, table_type, table_var, W1, b1, W2, b2)


# speedup vs baseline: 2.5414x; 1.0569x over previous
import functools

import jax
import jax.numpy as jnp
from jax import lax
from jax.experimental import pallas as pl
from jax.experimental.pallas import tpu as pltpu
from jax.experimental.pallas import tpu_sc as plsc

B = 16384
D = 64
D_TYPE = 32
D_VAR = 16
H = 128
V_TYPE = 100
V_VAR = 3

NC = 2
NS = 16
NW = NC * NS
B_PER_W = B // NW
GATHER_CHUNK = 128
N_CHUNK = B_PER_W // GATHER_CHUNK
LANES = 16

BLK = 2048
GRID = B // BLK

TBLK = 32768
HBLK = TBLK // 2
TGRID = (1000000 + TBLK - 1) // TBLK
NPAIR = TGRID * HBLK
TSHIFT = TBLK.bit_length() - 1


def _transpose_body(in_ref, out_ref):
    qb = TBLK // 4
    for q in range(2):
        out_ref[q * qb:(q + 1) * qb, 0:D] = (
            in_ref[:, q * qb:(q + 1) * qb].T)
        out_ref[q * qb:(q + 1) * qb, D:2 * D] = (
            in_ref[:, HBLK + q * qb:HBLK + (q + 1) * qb].T)


def _transpose_pack(table_t):
    return pl.pallas_call(
        _transpose_body,
        grid=(TGRID,),
        in_specs=[pl.BlockSpec((D, TBLK), lambda i: (0, i))],
        out_specs=pl.BlockSpec((HBLK, 2 * D), lambda i: (i, 0)),
        out_shape=jax.ShapeDtypeStruct((NPAIR, 2 * D), jnp.float32),
    )(table_t)


def _sc_gather(table, idx):
    mesh = plsc.VectorSubcoreMesh(core_axis_name="c", subcore_axis_name="s")

    @functools.partial(
        pl.kernel,
        mesh=mesh,
        out_type=jax.ShapeDtypeStruct((B, D), jnp.float32),
        scratch_types=[
            pltpu.VMEM((B_PER_W,), jnp.int32),
            pltpu.VMEM((B_PER_W,), jnp.int32),
            pltpu.VMEM((B_PER_W, D), jnp.float32),
            pltpu.SemaphoreType.DMA,
        ],
        compiler_params=pltpu.CompilerParams(use_tc_tiling_on_sc=False),
    )
    def k(table_hbm, idx_hbm, out_hbm, idx_v, j_v, rows_v, sem):
        wid = lax.axis_index("s") * NC + lax.axis_index("c")
        base = wid * B_PER_W
        pltpu.sync_copy(idx_hbm.at[pl.ds(base, B_PER_W)], idx_v)

        def transform(kk, _):
            v = idx_v[pl.ds(kk * LANES, LANES)]
            g = ((v >> TSHIFT) << (TSHIFT - 1)) + (v & (HBLK - 1))
            j_v[pl.ds(kk * LANES, LANES)] = (g << 1) | ((v >> (TSHIFT - 1)) & 1)
            return 0

        lax.fori_loop(0, B_PER_W // LANES, transform, 0)
        copies = []
        for c in range(N_CHUNK):
            copies.append(pltpu.async_copy(
                table_hbm.at[j_v.at[pl.ds(c * GATHER_CHUNK, GATHER_CHUNK)]],
                rows_v.at[pl.ds(c * GATHER_CHUNK, GATHER_CHUNK), :],
                sem,
            ))
        for cp in copies:
            cp.wait()
        pltpu.sync_copy(rows_v, out_hbm.at[pl.ds(base, B_PER_W)])

    return k(table, idx)


def _mlp_body(t_ref, v_ref, pe_ref, tt_ref, tv_ref, w1_ref, b1_ref, w2_ref,
              b2_ref, out_ref):
    tids = t_ref[0, 0, :]
    vids = v_ref[0, 0, :]
    pe = pe_ref[...]

    iota_t = lax.broadcasted_iota(jnp.int32, (BLK, V_TYPE), 1)
    onehot_t = (tids[:, None] == iota_t).astype(jnp.float32)
    te = jnp.dot(onehot_t, tt_ref[...], preferred_element_type=jnp.float32)

    ve = ((vids[:, None] == 0).astype(jnp.float32) * tv_ref[0:1, :]
          + (vids[:, None] == 1).astype(jnp.float32) * tv_ref[1:2, :]
          + (vids[:, None] == 2).astype(jnp.float32) * tv_ref[2:3, :])

    h = (jnp.dot(pe, w1_ref[0:D, :], preferred_element_type=jnp.float32)
         + jnp.dot(te, w1_ref[D:D + D_TYPE, :],
                   preferred_element_type=jnp.float32)
         + jnp.dot(ve, w1_ref[D + D_TYPE:D + D_TYPE + D_VAR, :],
                   preferred_element_type=jnp.float32)
         + b1_ref[:][None, :])
    h = jnp.maximum(h, 0.0)
    out_ref[...] = (jnp.dot(h, w2_ref[...], preferred_element_type=jnp.float32)
                    + b2_ref[:][None, :])


def _mlp(pe, t3, v3, table_type, table_var, W1, b1, W2, b2):
    return pl.pallas_call(
        _mlp_body,
        grid=(GRID,),
        in_specs=[
            pl.BlockSpec((1, 1, BLK), lambda i: (i, 0, 0)),
            pl.BlockSpec((1, 1, BLK), lambda i: (i, 0, 0)),
            pl.BlockSpec((BLK, D), lambda i: (i, 0)),
            pl.BlockSpec((V_TYPE, D_TYPE), lambda i: (0, 0)),
            pl.BlockSpec((V_VAR, D_VAR), lambda i: (0, 0)),
            pl.BlockSpec((D + D_TYPE + D_VAR, H), lambda i: (0, 0)),
            pl.BlockSpec((H,), lambda i: (0,)),
            pl.BlockSpec((H, D), lambda i: (0, 0)),
            pl.BlockSpec((D,), lambda i: (0,)),
        ],
        out_specs=pl.BlockSpec((BLK, D), lambda i: (i, 0)),
        out_shape=jax.ShapeDtypeStruct((B, D), jnp.float32),
    )(t3, v3, pe, table_type, table_var, W1, b1, W2, b2)


def kernel(product_id, product_type, variant_sellable, table_prod, table_type,
           table_var, W1, b1, W2, b2):
    pairs = _transpose_pack(table_prod.T)
    table_rm = jnp.reshape(pairs, (2 * NPAIR, D))
    pe = _sc_gather(table_rm, product_id)
    t3 = product_type.astype(jnp.int32).reshape(GRID, 1, BLK)
    v3 = variant_sellable.astype(jnp.int32).reshape(GRID, 1, BLK)
    return _mlp(pe, t3, v3, table_type, table_var, W1, b1, W2, b2)


# speedup vs baseline: 2.6542x; 1.0444x over previous
import functools

import jax
import jax.numpy as jnp
from jax import lax
from jax.experimental import pallas as pl
from jax.experimental.pallas import tpu as pltpu
from jax.experimental.pallas import tpu_sc as plsc

B = 16384
D = 64
D_TYPE = 32
D_VAR = 16
H = 128
V_TYPE = 100
V_VAR = 3

NC = 2
NS = 16
NW = NC * NS
B_PER_W = B // NW
GATHER_CHUNK = 128
N_CHUNK = B_PER_W // GATHER_CHUNK
LANES = 16

BLK = 4096
GRID = B // BLK

TBLK = 32768
HBLK = TBLK // 2
TGRID = (1000000 + TBLK - 1) // TBLK
NPAIR = TGRID * HBLK
TSHIFT = TBLK.bit_length() - 1


def _transpose_body(in_ref, out_ref):
    eye = (lax.broadcasted_iota(jnp.int32, (D, D), 0)
           == lax.broadcasted_iota(jnp.int32, (D, D), 1)).astype(jnp.float32)
    dn = (((0,), (0,)), ((), ()))
    qb = TBLK // 4
    for q in range(2):
        out_ref[q * qb:(q + 1) * qb, 0:D] = (
            in_ref[:, q * qb:(q + 1) * qb].T)
        out_ref[q * qb:(q + 1) * qb, D:2 * D] = lax.dot_general(
            in_ref[:, HBLK + q * qb:HBLK + (q + 1) * qb], eye, dn,
            preferred_element_type=jnp.float32)


def _transpose_pack(table_t):
    return pl.pallas_call(
        _transpose_body,
        grid=(TGRID,),
        in_specs=[pl.BlockSpec((D, TBLK), lambda i: (0, i))],
        out_specs=pl.BlockSpec((HBLK, 2 * D), lambda i: (i, 0)),
        out_shape=jax.ShapeDtypeStruct((NPAIR, 2 * D), jnp.float32),
    )(table_t)


def _sc_gather(table, idx):
    mesh = plsc.VectorSubcoreMesh(core_axis_name="c", subcore_axis_name="s")

    @functools.partial(
        pl.kernel,
        mesh=mesh,
        out_type=jax.ShapeDtypeStruct((B, D), jnp.float32),
        scratch_types=[
            pltpu.VMEM((B_PER_W,), jnp.int32),
            pltpu.VMEM((B_PER_W,), jnp.int32),
            pltpu.VMEM((B_PER_W, D), jnp.float32),
            pltpu.SemaphoreType.DMA,
        ],
        compiler_params=pltpu.CompilerParams(use_tc_tiling_on_sc=False),
    )
    def k(table_hbm, idx_hbm, out_hbm, idx_v, j_v, rows_v, sem):
        wid = lax.axis_index("s") * NC + lax.axis_index("c")
        base = wid * B_PER_W
        pltpu.sync_copy(idx_hbm.at[pl.ds(base, B_PER_W)], idx_v)

        def transform(kk, _):
            v = idx_v[pl.ds(kk * LANES, LANES)]
            g = ((v >> TSHIFT) << (TSHIFT - 1)) + (v & (HBLK - 1))
            j_v[pl.ds(kk * LANES, LANES)] = (g << 1) | ((v >> (TSHIFT - 1)) & 1)
            return 0

        lax.fori_loop(0, B_PER_W // LANES, transform, 0)
        copies = []
        for c in range(N_CHUNK):
            copies.append(pltpu.async_copy(
                table_hbm.at[j_v.at[pl.ds(c * GATHER_CHUNK, GATHER_CHUNK)]],
                rows_v.at[pl.ds(c * GATHER_CHUNK, GATHER_CHUNK), :],
                sem,
            ))
        for cp in copies:
            cp.wait()
        pltpu.sync_copy(rows_v, out_hbm.at[pl.ds(base, B_PER_W)])

    return k(table, idx)


def _mlp_body(t_ref, v_ref, pe_ref, ttT_ref, tvT_ref, w1_ref, b1c_ref,
              w2T_ref, b2c_ref, outT_ref):
    tids = t_ref[0, 0, :]
    vids = v_ref[0, 0, :]
    pe = pe_ref[...]

    iota_t = lax.broadcasted_iota(jnp.int32, (V_TYPE, BLK), 0)
    onehotT = (tids[None, :] == iota_t).astype(jnp.float32)
    teT = jnp.dot(ttT_ref[...], onehotT, preferred_element_type=jnp.float32)

    veT = ((vids[None, :] == 0).astype(jnp.float32) * tvT_ref[:, 0:1]
           + (vids[None, :] == 1).astype(jnp.float32) * tvT_ref[:, 1:2]
           + (vids[None, :] == 2).astype(jnp.float32) * tvT_ref[:, 2:3])

    dn0 = (((0,), (0,)), ((), ()))
    dn1 = (((0,), (1,)), ((), ()))
    hT = (lax.dot_general(w1_ref[0:D, :], pe, dn1,
                          preferred_element_type=jnp.float32)
          + lax.dot_general(w1_ref[D:D + D_TYPE, :], teT, dn0,
                            preferred_element_type=jnp.float32)
          + lax.dot_general(w1_ref[D + D_TYPE:D + D_TYPE + D_VAR, :], veT,
                            dn0, preferred_element_type=jnp.float32)
          + b1c_ref[...])
    hT = jnp.maximum(hT, 0.0)
    outT_ref[...] = (jnp.dot(w2T_ref[...], hT,
                             preferred_element_type=jnp.float32)
                     + b2c_ref[...])


def _mlp_t(pe, t3, v3, ttT, tvT, W1, b1c, W2T, b2c):
    return pl.pallas_call(
        _mlp_body,
        grid=(GRID,),
        in_specs=[
            pl.BlockSpec((1, 1, BLK), lambda i: (i, 0, 0)),
            pl.BlockSpec((1, 1, BLK), lambda i: (i, 0, 0)),
            pl.BlockSpec((BLK, D), lambda i: (i, 0)),
            pl.BlockSpec((D_TYPE, V_TYPE), lambda i: (0, 0)),
            pl.BlockSpec((D_VAR, V_VAR), lambda i: (0, 0)),
            pl.BlockSpec((D + D_TYPE + D_VAR, H), lambda i: (0, 0)),
            pl.BlockSpec((H, 1), lambda i: (0, 0)),
            pl.BlockSpec((D, H), lambda i: (0, 0)),
            pl.BlockSpec((D, 1), lambda i: (0, 0)),
        ],
        out_specs=pl.BlockSpec((D, BLK), lambda i: (0, i)),
        out_shape=jax.ShapeDtypeStruct((D, B), jnp.float32),
    )(t3, v3, pe, ttT, tvT, W1, b1c, W2T, b2c)


def kernel(product_id, product_type, variant_sellable, table_prod, table_type,
           table_var, W1, b1, W2, b2):
    pairs = _transpose_pack(table_prod.T)
    table_rm = jnp.reshape(pairs, (2 * NPAIR, D))
    pe = _sc_gather(table_rm, product_id)
    t3 = product_type.astype(jnp.int32).reshape(GRID, 1, BLK)
    v3 = variant_sellable.astype(jnp.int32).reshape(GRID, 1, BLK)
    outT = _mlp_t(pe, t3, v3, table_type.T, table_var.T, W1,
                  b1.reshape(H, 1), W2.T, b2.reshape(D, 1))
    return outT.T


# speedup vs baseline: 2.7503x; 1.0362x over previous
import functools

import jax
import jax.numpy as jnp
from jax import lax
from jax.experimental import pallas as pl
from jax.experimental.pallas import tpu as pltpu
from jax.experimental.pallas import tpu_sc as plsc

B = 16384
D = 64
D_TYPE = 32
D_VAR = 16
H = 128
V_TYPE = 100
V_VAR = 3

NC = 2
NS = 16
NW = NC * NS
B_PER_W = B // NW
GATHER_CHUNK = 128
N_CHUNK = B_PER_W // GATHER_CHUNK
LANES = 16

BLK = 4096
GRID = B // BLK

TBLK = 32768
HBLK = TBLK // 2
TGRID = (1000000 + TBLK - 1) // TBLK
NPAIR = TGRID * HBLK
TSHIFT = TBLK.bit_length() - 1


def _transpose_body(in_ref, out_ref):
    eye = (lax.broadcasted_iota(jnp.int32, (D, D), 0)
           == lax.broadcasted_iota(jnp.int32, (D, D), 1)).astype(jnp.float32)
    dn = (((0,), (0,)), ((), ()))
    qb = TBLK // 4
    for q in range(2):
        out_ref[q * qb:(q + 1) * qb, 0:D] = (
            in_ref[:, q * qb:(q + 1) * qb].T)
        out_ref[q * qb:(q + 1) * qb, D:2 * D] = lax.dot_general(
            in_ref[:, HBLK + q * qb:HBLK + (q + 1) * qb], eye, dn,
            preferred_element_type=jnp.float32)


def _transpose_pack(table_t):
    return pl.pallas_call(
        _transpose_body,
        grid=(TGRID,),
        in_specs=[pl.BlockSpec((D, TBLK), lambda i: (0, i))],
        out_specs=pl.BlockSpec((HBLK, 2 * D), lambda i: (i, 0)),
        out_shape=jax.ShapeDtypeStruct((NPAIR, 2 * D), jnp.float32),
    )(table_t)


def _sc_gather(table, idx):
    mesh = plsc.VectorSubcoreMesh(core_axis_name="c", subcore_axis_name="s")

    @functools.partial(
        pl.kernel,
        mesh=mesh,
        out_type=jax.ShapeDtypeStruct((B, 2 * D), jnp.float32),
        scratch_types=[
            pltpu.VMEM((B_PER_W,), jnp.int32),
            pltpu.VMEM((B_PER_W,), jnp.int32),
            pltpu.VMEM((B_PER_W, D), jnp.float32),
            pltpu.SemaphoreType.DMA,
        ],
        compiler_params=pltpu.CompilerParams(use_tc_tiling_on_sc=False),
    )
    def k(table_hbm, idx_hbm, out_hbm, idx_v, j_v, rows_v, sem):
        wid = lax.axis_index("s") * NC + lax.axis_index("c")
        base = wid * B_PER_W
        pltpu.sync_copy(idx_hbm.at[pl.ds(base, B_PER_W)], idx_v)

        def transform(kk, _):
            v = idx_v[pl.ds(kk * LANES, LANES)]
            g = ((v >> TSHIFT) << (TSHIFT - 1)) + (v & (HBLK - 1))
            j_v[pl.ds(kk * LANES, LANES)] = (g << 1) | ((v >> (TSHIFT - 1)) & 1)
            return 0

        lax.fori_loop(0, B_PER_W // LANES, transform, 0)
        copies = []
        for c in range(N_CHUNK):
            copies.append(pltpu.async_copy(
                table_hbm.at[j_v.at[pl.ds(c * GATHER_CHUNK, GATHER_CHUNK)]],
                rows_v.at[pl.ds(c * GATHER_CHUNK, GATHER_CHUNK), :],
                sem,
            ))
        for cp in copies:
            cp.wait()
        pltpu.sync_copy(rows_v,
                        out_hbm.at[pl.ds(base, B_PER_W), pl.ds(0, D)])

    return k(table, idx)


def _mlp_body(t_ref, v_ref, pe_ref, ttT_ref, tvT_ref, w1_ref, b1c_ref,
              w2T_ref, b2c_ref, outT_ref):
    tids = t_ref[0, 0, :]
    vids = v_ref[0, 0, :]
    pe = pe_ref[:, 0:D]

    iota_t = lax.broadcasted_iota(jnp.int32, (V_TYPE, BLK), 0)
    onehotT = (tids[None, :] == iota_t).astype(jnp.float32)
    teT = jnp.dot(ttT_ref[...], onehotT, preferred_element_type=jnp.float32)

    veT = ((vids[None, :] == 0).astype(jnp.float32) * tvT_ref[:, 0:1]
           + (vids[None, :] == 1).astype(jnp.float32) * tvT_ref[:, 1:2]
           + (vids[None, :] == 2).astype(jnp.float32) * tvT_ref[:, 2:3])

    dn0 = (((0,), (0,)), ((), ()))
    dn1 = (((0,), (1,)), ((), ()))
    hT = (lax.dot_general(w1_ref[0:D, :], pe, dn1,
                          preferred_element_type=jnp.float32)
          + lax.dot_general(w1_ref[D:D + D_TYPE, :], teT, dn0,
                            preferred_element_type=jnp.float32)
          + lax.dot_general(w1_ref[D + D_TYPE:D + D_TYPE + D_VAR, :], veT,
                            dn0, preferred_element_type=jnp.float32)
          + b1c_ref[...])
    hT = jnp.maximum(hT, 0.0)
    outT_ref[...] = (jnp.dot(w2T_ref[...], hT,
                             preferred_element_type=jnp.float32)
                     + b2c_ref[...])


def _mlp_t(pe, t3, v3, ttT, tvT, W1, b1c, W2T, b2c):
    return pl.pallas_call(
        _mlp_body,
        grid=(GRID,),
        in_specs=[
            pl.BlockSpec((1, 1, BLK), lambda i: (i, 0, 0)),
            pl.BlockSpec((1, 1, BLK), lambda i: (i, 0, 0)),
            pl.BlockSpec((BLK, 2 * D), lambda i: (i, 0)),
            pl.BlockSpec((D_TYPE, V_TYPE), lambda i: (0, 0)),
            pl.BlockSpec((D_VAR, V_VAR), lambda i: (0, 0)),
            pl.BlockSpec((D + D_TYPE + D_VAR, H), lambda i: (0, 0)),
            pl.BlockSpec((H, 1), lambda i: (0, 0)),
            pl.BlockSpec((D, H), lambda i: (0, 0)),
            pl.BlockSpec((D, 1), lambda i: (0, 0)),
        ],
        out_specs=pl.BlockSpec((D, BLK), lambda i: (0, i)),
        out_shape=jax.ShapeDtypeStruct((D, B), jnp.float32),
    )(t3, v3, pe, ttT, tvT, W1, b1c, W2T, b2c)


def kernel(product_id, product_type, variant_sellable, table_prod, table_type,
           table_var, W1, b1, W2, b2):
    pairs = _transpose_pack(table_prod.T)
    table_rm = jnp.reshape(pairs, (2 * NPAIR, D))
    pe = _sc_gather(table_rm, product_id)
    t3 = product_type.astype(jnp.int32).reshape(GRID, 1, BLK)
    v3 = variant_sellable.astype(jnp.int32).reshape(GRID, 1, BLK)
    outT = _mlp_t(pe, t3, v3, table_type.T, table_var.T, W1,
                  b1.reshape(H, 1), W2.T, b2.reshape(D, 1))
    return outT.T
